# baseline (speedup 1.0000x reference)
import dataclasses
import functools

import jax
import jax.numpy as jnp
from jax import lax
from jax.experimental import pallas as pl
from jax.experimental.pallas import tpu as pltpu
from jax.experimental.pallas import tpu_sc as plsc

N = 10000
E = 320000
D = 128

NC = 2
NS = 16
NW = NC * NS
CHUNK = 128
NCH = 80
E_PAD = NW * NCH * CHUNK
N_PAD = 10240
RPS = N_PAD // NS

_mesh = plsc.VectorSubcoreMesh(core_axis_name="c", subcore_axis_name="s")

_sc_params = pltpu.CompilerParams()
if "needs_layout_passes" in pltpu.CompilerParams.__dataclass_fields__:
    _sc_params = dataclasses.replace(_sc_params, needs_layout_passes=False)


HR = 128
EPW = NCH * CHUNK
_VEC = 16


@functools.partial(
    pl.kernel,
    mesh=_mesh,
    out_type=[jax.ShapeDtypeStruct((NC, HR, D), jnp.float32),
              jax.ShapeDtypeStruct((NC, HR, D), jnp.float32)],
    scratch_types=[
        pltpu.VMEM((EPW,), jnp.int32),
        pltpu.VMEM((EPW,), jnp.int32),
        pltpu.VMEM((HR, D), jnp.float32),
        pltpu.VMEM((HR, D), jnp.float32),
        pltpu.VMEM((1, HR), jnp.int32),
        pltpu.VMEM_SHARED((HR, D), jnp.float32),
        pltpu.VMEM_SHARED((HR, D), jnp.float32),
    ],
    compiler_params=_sc_params,
)
def _sc_degrees(src_hbm, dst_hbm, rows_hbm, z_hbm, od_hbm, id_hbm,
                src_v, dst_v, hs_v, hd_v, rows_v, od_acc, id_acc):
    c = lax.axis_index("c")
    s = lax.axis_index("s")
    w = c * NS + s
    r0 = s * (HR // NS)
    pltpu.sync_copy(z_hbm.at[pl.ds(0, HR)], hs_v)
    pltpu.sync_copy(z_hbm.at[pl.ds(0, HR)], hd_v)
    pltpu.sync_copy(z_hbm.at[pl.ds(r0, HR // NS)], od_acc.at[pl.ds(r0, HR // NS)])
    pltpu.sync_copy(z_hbm.at[pl.ds(r0, HR // NS)], id_acc.at[pl.ds(r0, HR // NS)])
    pltpu.sync_copy(rows_hbm, rows_v)
    pltpu.sync_copy(src_hbm.at[w], src_v)
    pltpu.sync_copy(dst_hbm.at[w], dst_v)

    one = jnp.full((_VEC,), 1.0, jnp.float32)

    @pl.loop(0, EPW // _VEC)
    def _(k):
        for idx_v, h_v in ((src_v, hs_v), (dst_v, hd_v)):
            n = idx_v[pl.ds(k * _VEC, _VEC)]
            plsc.addupdate_scatter(h_v, [n >> 7, n & 127], one)

    plsc.subcore_barrier()
    pltpu.sync_copy(hs_v, od_acc.at[rows_v.at[0]], add=True)
    pltpu.sync_copy(hd_v, id_acc.at[rows_v.at[0]], add=True)
    plsc.subcore_barrier()
    pltpu.sync_copy(od_acc.at[pl.ds(r0, HR // NS)],
                    od_hbm.at[c].at[pl.ds(r0, HR // NS)])
    pltpu.sync_copy(id_acc.at[pl.ds(r0, HR // NS)],
                    id_hbm.at[c].at[pl.ds(r0, HR // NS)])


STG = 16
NCH0 = 2 * NCH


def _agg_pipeline(h_hbm, src_w, dst_w, acc, sidx, didx, buf_a, buf_b,
                  sem_a, sem_b, sem_i, nch):
    nstg = nch // STG
    pltpu.async_copy(src_w.at[pl.ds(0, STG)], sidx.at[0], sem_i)
    pltpu.async_copy(dst_w.at[pl.ds(0, STG)], didx.at[0], sem_i)

    @pl.loop(0, nstg)
    def _(t):
        tm = t % 2
        pltpu.make_async_copy(src_w.at[pl.ds(t * STG, STG)],
                              sidx.at[tm], sem_i).wait()
        pltpu.make_async_copy(dst_w.at[pl.ds(t * STG, STG)],
                              didx.at[tm], sem_i).wait()

        @pl.when(t + 1 < nstg)
        def _():
            tn = (t + 1) % 2
            pltpu.async_copy(src_w.at[pl.ds((t + 1) * STG, STG)],
                             sidx.at[tn], sem_i)
            pltpu.async_copy(dst_w.at[pl.ds((t + 1) * STG, STG)],
                             didx.at[tn], sem_i)

        pltpu.async_copy(h_hbm.at[sidx.at[tm, 0]], buf_a, sem_a)
        pltpu.async_copy(h_hbm.at[sidx.at[tm, 1]], buf_b, sem_b)

        @pl.loop(0, STG, step=2)
        def _(j):
            pltpu.make_async_copy(h_hbm.at[sidx.at[tm, j]], buf_a, sem_a).wait()
            pltpu.sync_copy(buf_a, acc.at[didx.at[tm, j]], add=True)

            @pl.when(j + 2 < STG)
            def _():
                pltpu.async_copy(h_hbm.at[sidx.at[tm, j + 2]], buf_a, sem_a)

            pltpu.make_async_copy(h_hbm.at[sidx.at[tm, j + 1]], buf_b,
                                  sem_b).wait()
            pltpu.sync_copy(buf_b, acc.at[didx.at[tm, j + 1]], add=True)

            @pl.when(j + 3 < STG)
            def _():
                pltpu.async_copy(h_hbm.at[sidx.at[tm, j + 3]], buf_b, sem_b)


@functools.partial(
    pl.kernel,
    mesh=_mesh,
    out_type=jax.ShapeDtypeStruct((N_PAD, D), jnp.float32),
    scratch_types=[
        pltpu.VMEM((2, STG, CHUNK), jnp.int32),
        pltpu.VMEM((2, STG, CHUNK), jnp.int32),
        pltpu.VMEM((CHUNK, D), jnp.float32),
        pltpu.VMEM((CHUNK, D), jnp.float32),
        pltpu.VMEM_SHARED((N_PAD, D), jnp.float32),
        pltpu.SemaphoreType.DMA,
        pltpu.SemaphoreType.DMA,
        pltpu.SemaphoreType.DMA,
    ],
)
def _sc_aggregate(h_hbm, srcA, dstA, z_hbm, out_hbm,
                  sidx, didx, buf_a, buf_b, acc, sem_a, sem_b, sem_i):
    c = lax.axis_index("c")
    s = lax.axis_index("s")
    r0 = s * RPS

    @pl.when(c == 0)
    def _():
        pltpu.sync_copy(z_hbm.at[pl.ds(r0, RPS)], acc.at[pl.ds(r0, RPS)])
        plsc.subcore_barrier()
        _agg_pipeline(h_hbm, srcA.at[s], dstA.at[s], acc, sidx, didx,
                      buf_a, buf_b, sem_a, sem_b, sem_i, NCH0)
        plsc.subcore_barrier()
        pltpu.sync_copy(acc.at[pl.ds(r0, RPS)], out_hbm.at[pl.ds(r0, RPS)])


_BR = 512
_GRID = N_PAD // _BR


def _norm(ref):
    return lax.rsqrt(jnp.maximum(ref[...], 1.0))


def _scale_body(x_ref, od_ref, o_ref):
    o_ref[...] = x_ref[...] * _norm(od_ref)


def _mid_body(p_ref, od_ref, id_ref, w_ref, b_ref, o_ref):
    dst_n = _norm(id_ref)
    src_n = _norm(od_ref)
    agg = p_ref[...] * dst_n
    h = jnp.dot(agg, w_ref[...], preferred_element_type=jnp.float32)
    h = jnp.maximum(h + b_ref[...], 0.0) * src_n
    row = pl.program_id(0) * _BR + lax.broadcasted_iota(jnp.int32, (_BR, D), 0)
    o_ref[...] = jnp.where(row < N, h, 0.0)


def _final_body(p_ref, id_ref, w_ref, b_ref, o_ref):
    agg = p_ref[...] * _norm(id_ref)
    o_ref[...] = jnp.dot(agg, w_ref[...],
                         preferred_element_type=jnp.float32) + b_ref[...]


_deg_spec = pl.BlockSpec((_BR, 1), lambda i: (i, 0))
_part_spec = pl.BlockSpec((_BR, D), lambda i: (i, 0))
_row_spec = pl.BlockSpec((_BR, D), lambda i: (i, 0))
_w_spec = pl.BlockSpec((D, D), lambda i: (0, 0))
_b_spec = pl.BlockSpec((1, D), lambda i: (0, 0))
_out_struct = jax.ShapeDtypeStruct((N_PAD, D), jnp.float32)

_tc_scale = pl.pallas_call(
    _scale_body, grid=(_GRID,), in_specs=[_row_spec, _deg_spec],
    out_specs=_row_spec, out_shape=_out_struct)

_tc_mid = pl.pallas_call(
    _mid_body, grid=(_GRID,),
    in_specs=[_part_spec, _deg_spec, _deg_spec, _w_spec, _b_spec],
    out_specs=_row_spec, out_shape=_out_struct)

_tc_final = pl.pallas_call(
    _final_body, grid=(_GRID,),
    in_specs=[_part_spec, _deg_spec, _w_spec, _b_spec],
    out_specs=_row_spec, out_shape=_out_struct)


def kernel(inputs, edge_index, W1, b1, W2, b2):
    src = edge_index[0].astype(jnp.int32)
    dst = edge_index[1].astype(jnp.int32)
    pad = jnp.full((E_PAD - E,), N, jnp.int32)
    src_f = jnp.concatenate([src, pad])
    dst_f = jnp.concatenate([dst, pad])
    srcA = src_f.reshape(NS, NCH0, CHUNK)
    dstA = dst_f.reshape(NS, NCH0, CHUNK)

    zeros_d = jnp.zeros((N_PAD, D), jnp.float32)
    rows = jnp.arange(HR, dtype=jnp.int32).reshape(1, HR)
    x_pad = jnp.zeros((N_PAD, D), jnp.float32).at[:N].set(inputs)

    od, idg = _sc_degrees(src_f.reshape(NW, EPW), dst_f.reshape(NW, EPW),
                          rows, zeros_d)
    od_n = (od[0] + od[1]).reshape(HR * D)[:N_PAD].reshape(N_PAD, 1)
    id_n = (idg[0] + idg[1]).reshape(HR * D)[:N_PAD].reshape(N_PAD, 1)

    h1 = _tc_scale(x_pad, od_n)
    p1 = _sc_aggregate(h1, srcA, dstA, zeros_d)
    h2 = _tc_mid(p1, od_n, id_n, W1, b1.reshape(1, D))
    p2 = _sc_aggregate(h2, srcA, dstA, zeros_d)
    out = _tc_final(p2, id_n, W2, b2.reshape(1, D))
    return out[:N]


# speedup vs baseline: 3.5148x; 3.5148x over previous
import dataclasses
import functools

import jax
import jax.numpy as jnp
from jax import lax
from jax.experimental import pallas as pl
from jax.experimental.pallas import tpu as pltpu
from jax.experimental.pallas import tpu_sc as plsc

N = 10000
E = 320000
D = 128

NC = 2
NS = 16
NW = NC * NS
CHUNK = 128
NCH = 80
E_PAD = NW * NCH * CHUNK
N_PAD = 10240
RPS = N_PAD // NS

_mesh = plsc.VectorSubcoreMesh(core_axis_name="c", subcore_axis_name="s")

_sc_params = pltpu.CompilerParams()
if "needs_layout_passes" in pltpu.CompilerParams.__dataclass_fields__:
    _sc_params = dataclasses.replace(_sc_params, needs_layout_passes=False)


HR = 128
EPW = NCH * CHUNK
_VEC = 16


@functools.partial(
    pl.kernel,
    mesh=_mesh,
    out_type=[jax.ShapeDtypeStruct((NC, HR, D), jnp.float32),
              jax.ShapeDtypeStruct((NC, HR, D), jnp.float32)],
    scratch_types=[
        pltpu.VMEM((EPW,), jnp.int32),
        pltpu.VMEM((EPW,), jnp.int32),
        pltpu.VMEM((HR, D), jnp.float32),
        pltpu.VMEM((HR, D), jnp.float32),
        pltpu.VMEM((1, HR), jnp.int32),
        pltpu.VMEM_SHARED((HR, D), jnp.float32),
        pltpu.VMEM_SHARED((HR, D), jnp.float32),
    ],
    compiler_params=_sc_params,
)
def _sc_degrees(src_hbm, dst_hbm, rows_hbm, z_hbm, od_hbm, id_hbm,
                src_v, dst_v, hs_v, hd_v, rows_v, od_acc, id_acc):
    c = lax.axis_index("c")
    s = lax.axis_index("s")
    w = c * NS + s
    r0 = s * (HR // NS)
    pltpu.sync_copy(z_hbm.at[pl.ds(0, HR)], hs_v)
    pltpu.sync_copy(z_hbm.at[pl.ds(0, HR)], hd_v)
    pltpu.sync_copy(z_hbm.at[pl.ds(r0, HR // NS)], od_acc.at[pl.ds(r0, HR // NS)])
    pltpu.sync_copy(z_hbm.at[pl.ds(r0, HR // NS)], id_acc.at[pl.ds(r0, HR // NS)])
    pltpu.sync_copy(rows_hbm, rows_v)
    pltpu.sync_copy(src_hbm.at[w], src_v)
    pltpu.sync_copy(dst_hbm.at[w], dst_v)

    one = jnp.full((_VEC,), 1.0, jnp.float32)

    @pl.loop(0, EPW // _VEC)
    def _(k):
        for idx_v, h_v in ((src_v, hs_v), (dst_v, hd_v)):
            n = idx_v[pl.ds(k * _VEC, _VEC)]
            plsc.addupdate_scatter(h_v, [n >> 7, n & 127], one)

    plsc.subcore_barrier()
    pltpu.sync_copy(hs_v, od_acc.at[rows_v.at[0]], add=True)
    pltpu.sync_copy(hd_v, id_acc.at[rows_v.at[0]], add=True)
    plsc.subcore_barrier()
    pltpu.sync_copy(od_acc.at[pl.ds(r0, HR // NS)],
                    od_hbm.at[c].at[pl.ds(r0, HR // NS)])
    pltpu.sync_copy(id_acc.at[pl.ds(r0, HR // NS)],
                    id_hbm.at[c].at[pl.ds(r0, HR // NS)])


STG = 16


def _agg_pipeline(h_hbm, src_w, dst_w, acc, sidx, didx, buf_a, buf_b,
                  sem_a, sem_b, sem_i, nch):
    nstg = nch // STG
    pltpu.async_copy(src_w.at[pl.ds(0, STG)], sidx.at[0], sem_i)
    pltpu.async_copy(dst_w.at[pl.ds(0, STG)], didx.at[0], sem_i)

    @pl.loop(0, nstg)
    def _(t):
        tm = t % 2
        pltpu.make_async_copy(src_w.at[pl.ds(t * STG, STG)],
                              sidx.at[tm], sem_i).wait()
        pltpu.make_async_copy(dst_w.at[pl.ds(t * STG, STG)],
                              didx.at[tm], sem_i).wait()

        @pl.when(t + 1 < nstg)
        def _():
            tn = (t + 1) % 2
            pltpu.async_copy(src_w.at[pl.ds((t + 1) * STG, STG)],
                             sidx.at[tn], sem_i)
            pltpu.async_copy(dst_w.at[pl.ds((t + 1) * STG, STG)],
                             didx.at[tn], sem_i)

        pltpu.async_copy(h_hbm.at[sidx.at[tm, 0]], buf_a, sem_a)
        pltpu.async_copy(h_hbm.at[sidx.at[tm, 1]], buf_b, sem_b)

        @pl.loop(0, STG, step=2)
        def _(j):
            pltpu.make_async_copy(h_hbm.at[sidx.at[tm, j]], buf_a, sem_a).wait()
            pltpu.sync_copy(buf_a, acc.at[didx.at[tm, j]], add=True)

            @pl.when(j + 2 < STG)
            def _():
                pltpu.async_copy(h_hbm.at[sidx.at[tm, j + 2]], buf_a, sem_a)

            pltpu.make_async_copy(h_hbm.at[sidx.at[tm, j + 1]], buf_b,
                                  sem_b).wait()
            pltpu.sync_copy(buf_b, acc.at[didx.at[tm, j + 1]], add=True)

            @pl.when(j + 3 < STG)
            def _():
                pltpu.async_copy(h_hbm.at[sidx.at[tm, j + 3]], buf_b, sem_b)


@functools.partial(
    pl.kernel,
    mesh=_mesh,
    out_type=jax.ShapeDtypeStruct((NC, N_PAD, D), jnp.float32),
    scratch_types=[
        pltpu.VMEM((2, STG, CHUNK), jnp.int32),
        pltpu.VMEM((2, STG, CHUNK), jnp.int32),
        pltpu.VMEM((CHUNK, D), jnp.float32),
        pltpu.VMEM((CHUNK, D), jnp.float32),
        pltpu.VMEM_SHARED((N_PAD, D), jnp.float32),
        pltpu.SemaphoreType.DMA,
        pltpu.SemaphoreType.DMA,
        pltpu.SemaphoreType.DMA,
    ],
)
def _sc_aggregate(h_hbm, src_hbm, dst_hbm, z_hbm, out_hbm,
                  sidx, didx, buf_a, buf_b, acc, sem_a, sem_b, sem_i):
    c = lax.axis_index("c")
    s = lax.axis_index("s")
    w = c * NS + s
    r0 = s * RPS
    pltpu.sync_copy(z_hbm.at[pl.ds(r0, RPS)], acc.at[pl.ds(r0, RPS)])
    plsc.subcore_barrier()
    _agg_pipeline(h_hbm, src_hbm.at[w], dst_hbm.at[w], acc, sidx, didx,
                  buf_a, buf_b, sem_a, sem_b, sem_i, NCH)
    plsc.subcore_barrier()
    pltpu.sync_copy(acc.at[pl.ds(r0, RPS)], out_hbm.at[c].at[pl.ds(r0, RPS)])


_BR = 512
_GRID = N_PAD // _BR


def _norm(ref):
    return lax.rsqrt(jnp.maximum(ref[...], 1.0))


def _scale_body(x_ref, od_ref, o_ref):
    o_ref[...] = x_ref[...] * _norm(od_ref)


def _mid_body(p_ref, od_ref, id_ref, w_ref, b_ref, o_ref):
    dst_n = _norm(id_ref)
    src_n = _norm(od_ref)
    agg = (p_ref[0] + p_ref[1]) * dst_n
    h = jnp.dot(agg, w_ref[...], preferred_element_type=jnp.float32)
    h = jnp.maximum(h + b_ref[...], 0.0) * src_n
    row = pl.program_id(0) * _BR + lax.broadcasted_iota(jnp.int32, (_BR, D), 0)
    o_ref[...] = jnp.where(row < N, h, 0.0)


def _final_body(p_ref, id_ref, w_ref, b_ref, o_ref):
    agg = (p_ref[0] + p_ref[1]) * _norm(id_ref)
    o_ref[...] = jnp.dot(agg, w_ref[...],
                         preferred_element_type=jnp.float32) + b_ref[...]


_deg_spec = pl.BlockSpec((_BR, 1), lambda i: (i, 0))
_part_spec = pl.BlockSpec((NC, _BR, D), lambda i: (0, i, 0))
_row_spec = pl.BlockSpec((_BR, D), lambda i: (i, 0))
_w_spec = pl.BlockSpec((D, D), lambda i: (0, 0))
_b_spec = pl.BlockSpec((1, D), lambda i: (0, 0))
_out_struct = jax.ShapeDtypeStruct((N_PAD, D), jnp.float32)

_tc_scale = pl.pallas_call(
    _scale_body, grid=(_GRID,), in_specs=[_row_spec, _deg_spec],
    out_specs=_row_spec, out_shape=_out_struct)

_tc_mid = pl.pallas_call(
    _mid_body, grid=(_GRID,),
    in_specs=[_part_spec, _deg_spec, _deg_spec, _w_spec, _b_spec],
    out_specs=_row_spec, out_shape=_out_struct)

_tc_final = pl.pallas_call(
    _final_body, grid=(_GRID,),
    in_specs=[_part_spec, _deg_spec, _w_spec, _b_spec],
    out_specs=_row_spec, out_shape=_out_struct)


def kernel(inputs, edge_index, W1, b1, W2, b2):
    src = edge_index[0].astype(jnp.int32)
    dst = edge_index[1].astype(jnp.int32)
    pad = N + jnp.arange(E_PAD - E, dtype=jnp.int32) % (N_PAD - N)
    src_f = jnp.concatenate([src, pad])
    dst_f = jnp.concatenate([dst, pad])
    src_p = src_f.reshape(NW, NCH, CHUNK)
    dst_p = dst_f.reshape(NW, NCH, CHUNK)

    zeros_d = jnp.zeros((N_PAD, D), jnp.float32)
    rows = jnp.arange(HR, dtype=jnp.int32).reshape(1, HR)
    x_pad = jnp.zeros((N_PAD, D), jnp.float32).at[:N].set(inputs)

    od, idg = _sc_degrees(src_f.reshape(NW, EPW), dst_f.reshape(NW, EPW),
                          rows, zeros_d)
    od_n = (od[0] + od[1]).reshape(HR * D)[:N_PAD].reshape(N_PAD, 1)
    id_n = (idg[0] + idg[1]).reshape(HR * D)[:N_PAD].reshape(N_PAD, 1)

    h1 = _tc_scale(x_pad, od_n)
    p1 = _sc_aggregate(h1, src_p, dst_p, zeros_d)
    h2 = _tc_mid(p1, od_n, id_n, W1, b1.reshape(1, D))
    p2 = _sc_aggregate(h2, src_p, dst_p, zeros_d)
    out = _tc_final(p2, id_n, W2, b2.reshape(1, D))
    return out[:N]


# speedup vs baseline: 3.5728x; 1.0165x over previous
import dataclasses
import functools

import jax
import jax.numpy as jnp
from jax import lax
from jax.experimental import pallas as pl
from jax.experimental.pallas import tpu as pltpu
from jax.experimental.pallas import tpu_sc as plsc

N = 10000
E = 320000
D = 128

NC = 2
NS = 16
NW = NC * NS
CHUNK = 128
NCH = 80
E_PAD = NW * NCH * CHUNK
N_PAD = 10240
RPS = N_PAD // NS

_mesh = plsc.VectorSubcoreMesh(core_axis_name="c", subcore_axis_name="s")

_sc_params = pltpu.CompilerParams()
if "needs_layout_passes" in pltpu.CompilerParams.__dataclass_fields__:
    _sc_params = dataclasses.replace(_sc_params, needs_layout_passes=False)


HR = 128
EPW = NCH * CHUNK
_VEC = 16


@functools.partial(
    pl.kernel,
    mesh=_mesh,
    out_type=[jax.ShapeDtypeStruct((NC, HR, D), jnp.float32),
              jax.ShapeDtypeStruct((NC, HR, D), jnp.float32)],
    scratch_types=[
        pltpu.VMEM((EPW,), jnp.int32),
        pltpu.VMEM((EPW,), jnp.int32),
        pltpu.VMEM((HR, D), jnp.float32),
        pltpu.VMEM((HR, D), jnp.float32),
        pltpu.VMEM((1, HR), jnp.int32),
        pltpu.VMEM_SHARED((HR, D), jnp.float32),
        pltpu.VMEM_SHARED((HR, D), jnp.float32),
    ],
    compiler_params=_sc_params,
)
def _sc_degrees(src_hbm, dst_hbm, rows_hbm, z_hbm, od_hbm, id_hbm,
                src_v, dst_v, hs_v, hd_v, rows_v, od_acc, id_acc):
    c = lax.axis_index("c")
    s = lax.axis_index("s")
    w = c * NS + s
    r0 = s * (HR // NS)
    pltpu.sync_copy(z_hbm.at[pl.ds(0, HR)], hs_v)
    pltpu.sync_copy(z_hbm.at[pl.ds(0, HR)], hd_v)
    pltpu.sync_copy(z_hbm.at[pl.ds(r0, HR // NS)], od_acc.at[pl.ds(r0, HR // NS)])
    pltpu.sync_copy(z_hbm.at[pl.ds(r0, HR // NS)], id_acc.at[pl.ds(r0, HR // NS)])
    pltpu.sync_copy(rows_hbm, rows_v)
    pltpu.sync_copy(src_hbm.at[w], src_v)
    pltpu.sync_copy(dst_hbm.at[w], dst_v)

    one = jnp.full((_VEC,), 1.0, jnp.float32)

    @pl.loop(0, EPW // _VEC)
    def _(k):
        for idx_v, h_v in ((src_v, hs_v), (dst_v, hd_v)):
            n = idx_v[pl.ds(k * _VEC, _VEC)]
            plsc.addupdate_scatter(h_v, [n >> 7, n & 127], one)

    plsc.subcore_barrier()
    pltpu.sync_copy(hs_v, od_acc.at[rows_v.at[0]], add=True)
    pltpu.sync_copy(hd_v, id_acc.at[rows_v.at[0]], add=True)
    plsc.subcore_barrier()
    pltpu.sync_copy(od_acc.at[pl.ds(r0, HR // NS)],
                    od_hbm.at[c].at[pl.ds(r0, HR // NS)])
    pltpu.sync_copy(id_acc.at[pl.ds(r0, HR // NS)],
                    id_hbm.at[c].at[pl.ds(r0, HR // NS)])


STG = 16


def _agg_pipeline(h_hbm, src_w, dst_w, acc, sidx, didx, buf_a, buf_b,
                  sem_a, sem_b, sem_i, nch):
    nstg = nch // STG
    pltpu.async_copy(src_w.at[pl.ds(0, STG)], sidx.at[0], sem_i)
    pltpu.async_copy(dst_w.at[pl.ds(0, STG)], didx.at[0], sem_i)

    @pl.loop(0, nstg)
    def _(t):
        tm = t % 2
        pltpu.make_async_copy(src_w.at[pl.ds(t * STG, STG)],
                              sidx.at[tm], sem_i).wait()
        pltpu.make_async_copy(dst_w.at[pl.ds(t * STG, STG)],
                              didx.at[tm], sem_i).wait()

        @pl.when(t + 1 < nstg)
        def _():
            tn = (t + 1) % 2
            pltpu.async_copy(src_w.at[pl.ds((t + 1) * STG, STG)],
                             sidx.at[tn], sem_i)
            pltpu.async_copy(dst_w.at[pl.ds((t + 1) * STG, STG)],
                             didx.at[tn], sem_i)

        pltpu.async_copy(h_hbm.at[sidx.at[tm, 0]], buf_a, sem_a)
        pltpu.async_copy(h_hbm.at[sidx.at[tm, 1]], buf_b, sem_b)

        @pl.loop(0, STG, step=2)
        def _(j):
            pltpu.make_async_copy(h_hbm.at[sidx.at[tm, j]], buf_a, sem_a).wait()
            pltpu.sync_copy(buf_a, acc.at[didx.at[tm, j]], add=True)

            @pl.when(j + 2 < STG)
            def _():
                pltpu.async_copy(h_hbm.at[sidx.at[tm, j + 2]], buf_a, sem_a)

            pltpu.make_async_copy(h_hbm.at[sidx.at[tm, j + 1]], buf_b,
                                  sem_b).wait()
            pltpu.sync_copy(buf_b, acc.at[didx.at[tm, j + 1]], add=True)

            @pl.when(j + 3 < STG)
            def _():
                pltpu.async_copy(h_hbm.at[sidx.at[tm, j + 3]], buf_b, sem_b)


@functools.partial(
    pl.kernel,
    mesh=_mesh,
    out_type=jax.ShapeDtypeStruct((NC, N_PAD, D), jnp.float32),
    scratch_types=[
        pltpu.VMEM((2, STG, CHUNK), jnp.int32),
        pltpu.VMEM((2, STG, CHUNK), jnp.int32),
        pltpu.VMEM((CHUNK, D), jnp.float32),
        pltpu.VMEM((CHUNK, D), jnp.float32),
        pltpu.VMEM_SHARED((N_PAD, D), jnp.float32),
        pltpu.SemaphoreType.DMA,
        pltpu.SemaphoreType.DMA,
        pltpu.SemaphoreType.DMA,
    ],
)
def _sc_aggregate(h_hbm, src_hbm, dst_hbm, z_hbm, out_hbm,
                  sidx, didx, buf_a, buf_b, acc, sem_a, sem_b, sem_i):
    c = lax.axis_index("c")
    s = lax.axis_index("s")
    w = c * NS + s
    r0 = s * RPS
    pltpu.sync_copy(z_hbm.at[pl.ds(r0, RPS)], acc.at[pl.ds(r0, RPS)])
    plsc.subcore_barrier()
    _agg_pipeline(h_hbm, src_hbm.at[w], dst_hbm.at[w], acc, sidx, didx,
                  buf_a, buf_b, sem_a, sem_b, sem_i, NCH)
    plsc.subcore_barrier()
    pltpu.sync_copy(acc.at[pl.ds(r0, RPS)], out_hbm.at[c].at[pl.ds(r0, RPS)])


_BR = 512
_GRID = N_PAD // _BR


def _norm(ref):
    return lax.rsqrt(jnp.maximum(ref[...], 1.0))


def _scale_body(x_ref, od_ref, o_ref):
    row = pl.program_id(0) * _BR + lax.broadcasted_iota(jnp.int32, (_BR, D), 0)
    o_ref[...] = jnp.where(row < N, x_ref[...], 0.0) * _norm(od_ref)


def _mid_body(p_ref, od_ref, id_ref, w_ref, b_ref, o_ref):
    dst_n = _norm(id_ref)
    src_n = _norm(od_ref)
    agg = (p_ref[0] + p_ref[1]) * dst_n
    h = jnp.dot(agg, w_ref[...], preferred_element_type=jnp.float32)
    h = jnp.maximum(h + b_ref[...], 0.0) * src_n
    row = pl.program_id(0) * _BR + lax.broadcasted_iota(jnp.int32, (_BR, D), 0)
    o_ref[...] = jnp.where(row < N, h, 0.0)


def _final_body(p_ref, id_ref, w_ref, b_ref, o_ref):
    agg = (p_ref[0] + p_ref[1]) * _norm(id_ref)
    o_ref[...] = jnp.dot(agg, w_ref[...],
                         preferred_element_type=jnp.float32) + b_ref[...]


_deg_spec = pl.BlockSpec((_BR, 1), lambda i: (i, 0))
_part_spec = pl.BlockSpec((NC, _BR, D), lambda i: (0, i, 0))
_row_spec = pl.BlockSpec((_BR, D), lambda i: (i, 0))
_w_spec = pl.BlockSpec((D, D), lambda i: (0, 0))
_b_spec = pl.BlockSpec((1, D), lambda i: (0, 0))
_out_struct = jax.ShapeDtypeStruct((N_PAD, D), jnp.float32)

_tc_scale = pl.pallas_call(
    _scale_body, grid=(_GRID,), in_specs=[_row_spec, _deg_spec],
    out_specs=_row_spec, out_shape=_out_struct)

_tc_mid = pl.pallas_call(
    _mid_body, grid=(_GRID,),
    in_specs=[_part_spec, _deg_spec, _deg_spec, _w_spec, _b_spec],
    out_specs=_row_spec, out_shape=_out_struct)

_tc_final = pl.pallas_call(
    _final_body, grid=(_GRID,),
    in_specs=[_part_spec, _deg_spec, _w_spec, _b_spec],
    out_specs=_row_spec,
    out_shape=jax.ShapeDtypeStruct((N, D), jnp.float32))


def kernel(inputs, edge_index, W1, b1, W2, b2):
    src = edge_index[0].astype(jnp.int32)
    dst = edge_index[1].astype(jnp.int32)
    pad = N + jnp.arange(E_PAD - E, dtype=jnp.int32) % (N_PAD - N)
    src_f = jnp.concatenate([src, pad])
    dst_f = jnp.concatenate([dst, pad])
    src_p = src_f.reshape(NW, NCH, CHUNK)
    dst_p = dst_f.reshape(NW, NCH, CHUNK)

    zeros_d = jnp.zeros((N_PAD, D), jnp.float32)
    rows = jnp.arange(HR, dtype=jnp.int32).reshape(1, HR)

    od, idg = _sc_degrees(src_f.reshape(NW, EPW), dst_f.reshape(NW, EPW),
                          rows, zeros_d)
    od_n = (od[0] + od[1]).reshape(HR * D)[:N_PAD].reshape(N_PAD, 1)
    id_n = (idg[0] + idg[1]).reshape(HR * D)[:N_PAD].reshape(N_PAD, 1)

    h1 = _tc_scale(inputs, od_n)
    p1 = _sc_aggregate(h1, src_p, dst_p, zeros_d)
    h2 = _tc_mid(p1, od_n, id_n, W1, b1.reshape(1, D))
    p2 = _sc_aggregate(h2, src_p, dst_p, zeros_d)
    return _tc_final(p2, id_n, W2, b2.reshape(1, D))


# speedup vs baseline: 3.7376x; 1.0461x over previous
import dataclasses
import functools

import jax
import jax.numpy as jnp
from jax import lax
from jax.experimental import pallas as pl
from jax.experimental.pallas import tpu as pltpu
from jax.experimental.pallas import tpu_sc as plsc

N = 10000
E = 320000
D = 128

NC = 2
NS = 16
NW = NC * NS
CHUNK = 128
NCH = 80
E_PAD = NW * NCH * CHUNK
N_PAD = 10240
RPS = N_PAD // NS

_mesh = plsc.VectorSubcoreMesh(core_axis_name="c", subcore_axis_name="s")

_sc_params = pltpu.CompilerParams()
if "needs_layout_passes" in pltpu.CompilerParams.__dataclass_fields__:
    _sc_params = dataclasses.replace(_sc_params, needs_layout_passes=False)


HR = 128
EPW = NCH * CHUNK
_VEC = 16


@functools.partial(
    pl.kernel,
    mesh=_mesh,
    out_type=[jax.ShapeDtypeStruct((NC, HR, D), jnp.float32),
              jax.ShapeDtypeStruct((NC, HR, D), jnp.float32)],
    scratch_types=[
        pltpu.VMEM((EPW,), jnp.int32),
        pltpu.VMEM((EPW,), jnp.int32),
        pltpu.VMEM((HR, D), jnp.float32),
        pltpu.VMEM((HR, D), jnp.float32),
        pltpu.VMEM((1, HR), jnp.int32),
        pltpu.VMEM_SHARED((HR, D), jnp.float32),
        pltpu.VMEM_SHARED((HR, D), jnp.float32),
    ],
    compiler_params=_sc_params,
)
def _sc_degrees(src_hbm, dst_hbm, rows_hbm, z_hbm, od_hbm, id_hbm,
                src_v, dst_v, hs_v, hd_v, rows_v, od_acc, id_acc):
    c = lax.axis_index("c")
    s = lax.axis_index("s")
    w = c * NS + s
    r0 = s * (HR // NS)
    pltpu.sync_copy(z_hbm.at[pl.ds(0, HR)], hs_v)
    pltpu.sync_copy(z_hbm.at[pl.ds(0, HR)], hd_v)
    pltpu.sync_copy(z_hbm.at[pl.ds(r0, HR // NS)], od_acc.at[pl.ds(r0, HR // NS)])
    pltpu.sync_copy(z_hbm.at[pl.ds(r0, HR // NS)], id_acc.at[pl.ds(r0, HR // NS)])
    pltpu.sync_copy(rows_hbm, rows_v)
    pltpu.sync_copy(src_hbm.at[w], src_v)
    pltpu.sync_copy(dst_hbm.at[w], dst_v)

    one = jnp.full((_VEC,), 1.0, jnp.float32)

    @pl.loop(0, EPW // _VEC)
    def _(k):
        for idx_v, h_v in ((src_v, hs_v), (dst_v, hd_v)):
            n = idx_v[pl.ds(k * _VEC, _VEC)]
            plsc.addupdate_scatter(h_v, [n >> 7, n & 127], one)

    plsc.subcore_barrier()
    pltpu.sync_copy(hs_v, od_acc.at[rows_v.at[0]], add=True)
    pltpu.sync_copy(hd_v, id_acc.at[rows_v.at[0]], add=True)
    plsc.subcore_barrier()
    pltpu.sync_copy(od_acc.at[pl.ds(r0, HR // NS)],
                    od_hbm.at[c].at[pl.ds(r0, HR // NS)])
    pltpu.sync_copy(id_acc.at[pl.ds(r0, HR // NS)],
                    id_hbm.at[c].at[pl.ds(r0, HR // NS)])


STG = 16


def _agg_pipeline(h_hbm, src_w, dst_w, acc, sidx, didx, buf_a, buf_b,
                  sem_a, sem_b, sem_i, nch):
    nstg = nch // STG
    pltpu.async_copy(src_w.at[pl.ds(0, STG)], sidx.at[0], sem_i)
    pltpu.async_copy(dst_w.at[pl.ds(0, STG)], didx.at[0], sem_i)

    @pl.loop(0, nstg)
    def _(t):
        tm = t % 2
        pltpu.make_async_copy(src_w.at[pl.ds(t * STG, STG)],
                              sidx.at[tm], sem_i).wait()
        pltpu.make_async_copy(dst_w.at[pl.ds(t * STG, STG)],
                              didx.at[tm], sem_i).wait()

        @pl.when(t + 1 < nstg)
        def _():
            tn = (t + 1) % 2
            pltpu.async_copy(src_w.at[pl.ds((t + 1) * STG, STG)],
                             sidx.at[tn], sem_i)
            pltpu.async_copy(dst_w.at[pl.ds((t + 1) * STG, STG)],
                             didx.at[tn], sem_i)

        pltpu.async_copy(h_hbm.at[sidx.at[tm, 0]], buf_a, sem_a)
        pltpu.async_copy(h_hbm.at[sidx.at[tm, 1]], buf_b, sem_b)

        @pl.loop(0, STG, step=2)
        def _(j):
            pltpu.make_async_copy(h_hbm.at[sidx.at[tm, j]], buf_a, sem_a).wait()
            pltpu.sync_copy(buf_a, acc.at[didx.at[tm, j]], add=True)

            @pl.when(j + 2 < STG)
            def _():
                pltpu.async_copy(h_hbm.at[sidx.at[tm, j + 2]], buf_a, sem_a)

            pltpu.make_async_copy(h_hbm.at[sidx.at[tm, j + 1]], buf_b,
                                  sem_b).wait()
            pltpu.sync_copy(buf_b, acc.at[didx.at[tm, j + 1]], add=True)

            @pl.when(j + 3 < STG)
            def _():
                pltpu.async_copy(h_hbm.at[sidx.at[tm, j + 3]], buf_b, sem_b)


@functools.partial(
    pl.kernel,
    mesh=_mesh,
    out_type=jax.ShapeDtypeStruct((NC, N_PAD, D), jnp.float32),
    scratch_types=[
        pltpu.VMEM((2, STG, CHUNK), jnp.int32),
        pltpu.VMEM((2, STG, CHUNK), jnp.int32),
        pltpu.VMEM((CHUNK, D), jnp.float32),
        pltpu.VMEM((CHUNK, D), jnp.float32),
        pltpu.VMEM_SHARED((N_PAD, D), jnp.float32),
        pltpu.SemaphoreType.DMA,
        pltpu.SemaphoreType.DMA,
        pltpu.SemaphoreType.DMA,
    ],
)
def _sc_aggregate(h_hbm, src_hbm, dst_hbm, z_hbm, out_hbm,
                  sidx, didx, buf_a, buf_b, acc, sem_a, sem_b, sem_i):
    c = lax.axis_index("c")
    s = lax.axis_index("s")
    w = c * NS + s
    r0 = s * RPS
    pltpu.sync_copy(z_hbm.at[pl.ds(r0, RPS)], acc.at[pl.ds(r0, RPS)])
    plsc.subcore_barrier()
    _agg_pipeline(h_hbm, src_hbm.at[w], dst_hbm.at[w], acc, sidx, didx,
                  buf_a, buf_b, sem_a, sem_b, sem_i, NCH)
    plsc.subcore_barrier()
    pltpu.sync_copy(acc.at[pl.ds(r0, RPS)], out_hbm.at[c].at[pl.ds(r0, RPS)])


_BR = 1024
_GRID = N_PAD // _BR


def _norm(ref):
    return lax.rsqrt(jnp.maximum(ref[...], 1.0))


def _scale_body(x_ref, od_ref, o_ref):
    row = pl.program_id(0) * _BR + lax.broadcasted_iota(jnp.int32, (_BR, D), 0)
    o_ref[...] = jnp.where(row < N, x_ref[...], 0.0) * _norm(od_ref)


def _mid_body(p_ref, od_ref, id_ref, w_ref, b_ref, o_ref):
    dst_n = _norm(id_ref)
    src_n = _norm(od_ref)
    agg = (p_ref[0] + p_ref[1]) * dst_n
    h = jnp.dot(agg, w_ref[...], preferred_element_type=jnp.float32)
    h = jnp.maximum(h + b_ref[...], 0.0) * src_n
    row = pl.program_id(0) * _BR + lax.broadcasted_iota(jnp.int32, (_BR, D), 0)
    o_ref[...] = jnp.where(row < N, h, 0.0)


def _final_body(p_ref, id_ref, w_ref, b_ref, o_ref):
    agg = (p_ref[0] + p_ref[1]) * _norm(id_ref)
    o_ref[...] = jnp.dot(agg, w_ref[...],
                         preferred_element_type=jnp.float32) + b_ref[...]


_deg_spec = pl.BlockSpec((_BR, 1), lambda i: (i, 0))
_part_spec = pl.BlockSpec((NC, _BR, D), lambda i: (0, i, 0))
_row_spec = pl.BlockSpec((_BR, D), lambda i: (i, 0))
_w_spec = pl.BlockSpec((D, D), lambda i: (0, 0))
_b_spec = pl.BlockSpec((1, D), lambda i: (0, 0))
_out_struct = jax.ShapeDtypeStruct((N_PAD, D), jnp.float32)

_tc_scale = pl.pallas_call(
    _scale_body, grid=(_GRID,), in_specs=[_row_spec, _deg_spec],
    out_specs=_row_spec, out_shape=_out_struct)

_tc_mid = pl.pallas_call(
    _mid_body, grid=(_GRID,),
    in_specs=[_part_spec, _deg_spec, _deg_spec, _w_spec, _b_spec],
    out_specs=_row_spec, out_shape=_out_struct)

_tc_final = pl.pallas_call(
    _final_body, grid=(_GRID,),
    in_specs=[_part_spec, _deg_spec, _w_spec, _b_spec],
    out_specs=_row_spec,
    out_shape=jax.ShapeDtypeStruct((N, D), jnp.float32))


def kernel(inputs, edge_index, W1, b1, W2, b2):
    src = edge_index[0].astype(jnp.int32)
    dst = edge_index[1].astype(jnp.int32)
    pad = N + jnp.arange(E_PAD - E, dtype=jnp.int32) % (N_PAD - N)
    src_f = jnp.concatenate([src, pad])
    dst_f = jnp.concatenate([dst, pad])
    src_p = src_f.reshape(NW, NCH, CHUNK)
    dst_p = dst_f.reshape(NW, NCH, CHUNK)

    zeros_d = jnp.zeros((N_PAD, D), jnp.float32)
    rows = jnp.arange(HR, dtype=jnp.int32).reshape(1, HR)

    od, idg = _sc_degrees(src_f.reshape(NW, EPW), dst_f.reshape(NW, EPW),
                          rows, zeros_d)
    od_n = (od[0] + od[1]).reshape(HR * D)[:N_PAD].reshape(N_PAD, 1)
    id_n = (idg[0] + idg[1]).reshape(HR * D)[:N_PAD].reshape(N_PAD, 1)

    h1 = _tc_scale(inputs, od_n)
    p1 = _sc_aggregate(h1, src_p, dst_p, zeros_d)
    h2 = _tc_mid(p1, od_n, id_n, W1, b1.reshape(1, D))
    p2 = _sc_aggregate(h2, src_p, dst_p, zeros_d)
    return _tc_final(p2, id_n, W2, b2.reshape(1, D))
